# baseline (speedup 1.0000x reference)
import jax
import jax.numpy as jnp
from jax.experimental import pallas as pl
from jax.experimental.pallas import tpu as pltpu


def _copy_block(in_ref, out_ref):
    out_ref[...] = in_ref[...]


def kernel(tokens, embedding_weight):
    seq_len = tokens.shape[1]
    _, d_model = embedding_weight.shape
    block = 1024
    out = pl.pallas_call(
        _copy_block,
        grid=(seq_len // block,),
        in_specs=[pl.BlockSpec((block, d_model), lambda i: (i, 0))],
        out_specs=pl.BlockSpec((block, d_model), lambda i: (i, 0)),
        out_shape=jax.ShapeDtypeStruct((seq_len, d_model), embedding_weight.dtype),
        compiler_params=pltpu.CompilerParams(dimension_semantics=("parallel",)),
    )(embedding_weight)
    return out[None]
